# Initial kernel scaffold; baseline (speedup 1.0000x reference)
import jax
import jax.numpy as jnp
from jax.experimental import pallas as pl


def kernel(token_ids, weights):
    raise NotImplementedError("write your pallas kernel here")


# speedup vs baseline: 1.0942x; 1.0942x over previous
import functools

import jax
import jax.numpy as jnp
from jax import lax
from jax.experimental import pallas as pl
from jax.experimental.pallas import tpu as pltpu
from jax.experimental.pallas import tpu_sc as plsc

NUM_EMBEDDINGS = 1000000
EMBEDDING_DIM = 32
BATCH = 16384
HIST = 50

_TOTAL = BATCH * HIST
_NW = 32
_PER_W = _TOTAL // _NW
_CHUNK = 1024
_NCHUNK = _PER_W // _CHUNK


@functools.partial(
    pl.kernel,
    mesh=plsc.VectorSubcoreMesh(core_axis_name="c", subcore_axis_name="s"),
    out_type=jax.ShapeDtypeStruct((_TOTAL, EMBEDDING_DIM), jnp.float32),
    scratch_types=[
        pltpu.VMEM((_CHUNK,), jnp.int32),
        pltpu.VMEM((_CHUNK, EMBEDDING_DIM), jnp.float32),
        pltpu.SemaphoreType.DMA,
    ],
    compiler_params=pltpu.CompilerParams(use_tc_tiling_on_sc=False),
)
def _gather_kernel(ids_hbm, table_hbm, out_hbm, idx_v, rows_v, sem):
    wid = lax.axis_index("s") * 2 + lax.axis_index("c")
    base = wid * _PER_W

    def body(i, carry):
        off = base + i * _CHUNK
        pltpu.sync_copy(ids_hbm.at[pl.ds(off, _CHUNK)], idx_v)
        pltpu.async_copy(table_hbm.at[idx_v], rows_v, sem).wait()
        pltpu.sync_copy(rows_v, out_hbm.at[pl.ds(off, _CHUNK)])
        return carry

    lax.fori_loop(0, _NCHUNK, body, 0)


def kernel(token_ids, weights):
    flat_ids = token_ids.reshape(_TOTAL)
    out = _gather_kernel(flat_ids, weights)
    return out.reshape(BATCH, HIST, EMBEDDING_DIM)


# speedup vs baseline: 1.1138x; 1.0179x over previous
import functools

import jax
import jax.numpy as jnp
from jax import lax
from jax.experimental import pallas as pl
from jax.experimental.pallas import tpu as pltpu
from jax.experimental.pallas import tpu_sc as plsc

NUM_EMBEDDINGS = 1000000
EMBEDDING_DIM = 32
BATCH = 16384
HIST = 50

_TOTAL = BATCH * HIST
_NW = 32
_PER_W = _TOTAL // _NW
_CHUNK = 1600
_NCHUNK = _PER_W // _CHUNK


@functools.partial(
    pl.kernel,
    mesh=plsc.VectorSubcoreMesh(core_axis_name="c", subcore_axis_name="s"),
    out_type=jax.ShapeDtypeStruct((_TOTAL, EMBEDDING_DIM), jnp.float32),
    scratch_types=[
        pltpu.VMEM((_PER_W,), jnp.int32),
        pltpu.VMEM((_CHUNK, EMBEDDING_DIM), jnp.float32),
        pltpu.VMEM((_CHUNK, EMBEDDING_DIM), jnp.float32),
        pltpu.SemaphoreType.DMA,
        pltpu.SemaphoreType.DMA,
        pltpu.SemaphoreType.DMA,
    ],
    compiler_params=pltpu.CompilerParams(use_tc_tiling_on_sc=False),
)
def _gather_kernel(ids_hbm, table_hbm, out_hbm, idx_all, rows0, rows1,
                   sem_g, sem_s0, sem_s1):
    wid = lax.axis_index("s") * 2 + lax.axis_index("c")
    base = wid * _PER_W
    pltpu.sync_copy(ids_hbm.at[pl.ds(base, _PER_W)], idx_all)

    def g_start(i, buf):
        pltpu.make_async_copy(
            table_hbm.at[idx_all.at[pl.ds(i * _CHUNK, _CHUNK)]], buf, sem_g
        ).start()

    def g_wait(buf):
        pltpu.make_async_copy(
            table_hbm.at[idx_all.at[pl.ds(0, _CHUNK)]], buf, sem_g
        ).wait()

    def s_start(i, buf, sem):
        pltpu.make_async_copy(
            buf, out_hbm.at[pl.ds(base + i * _CHUNK, _CHUNK)], sem
        ).start()

    def s_wait(buf, sem):
        pltpu.make_async_copy(
            buf, out_hbm.at[pl.ds(base, _CHUNK)], sem
        ).wait()

    g_start(0, rows0)
    g_start(1, rows1)
    g_wait(rows0)
    s_start(0, rows0, sem_s0)

    def pair(gi, carry):
        i = 1 + 2 * gi
        s_wait(rows0, sem_s0)
        g_start(i + 1, rows0)
        g_wait(rows1)
        s_start(i, rows1, sem_s1)
        s_wait(rows1, sem_s1)
        g_start(i + 2, rows1)
        g_wait(rows0)
        s_start(i + 1, rows0, sem_s0)
        return carry

    lax.fori_loop(0, (_NCHUNK - 2) // 2, pair, 0)

    s_wait(rows0, sem_s0)
    g_wait(rows1)
    s_start(_NCHUNK - 1, rows1, sem_s1)
    s_wait(rows1, sem_s1)


def kernel(token_ids, weights):
    flat_ids = token_ids.reshape(_TOTAL)
    out = _gather_kernel(flat_ids, weights)
    return out.reshape(BATCH, HIST, EMBEDDING_DIM)


# speedup vs baseline: 2.0713x; 1.8597x over previous
import functools

import jax
import jax.numpy as jnp
from jax import lax
from jax.experimental import pallas as pl
from jax.experimental.pallas import tpu as pltpu
from jax.experimental.pallas import tpu_sc as plsc

NUM_EMBEDDINGS = 1000000
EMBEDDING_DIM = 32
BATCH = 16384
HIST = 50

_TOTAL = BATCH * HIST
_NW = 32
_BPW = BATCH // _NW
_PER_W = _BPW * HIST
_BLK = 128
_NBLK = _PER_W // _BLK


@functools.partial(
    pl.kernel,
    mesh=plsc.VectorSubcoreMesh(core_axis_name="c", subcore_axis_name="s"),
    out_type=jax.ShapeDtypeStruct((HIST, 4, BATCH // _BLK, 8 * _BLK), jnp.float32),
    scratch_types=[
        pltpu.VMEM((_PER_W,), jnp.int32),
        pltpu.VMEM((_BLK,), jnp.int32),
        pltpu.VMEM((_BLK,), jnp.int32),
        pltpu.VMEM((_BLK, EMBEDDING_DIM), jnp.float32),
        pltpu.VMEM((_BLK, EMBEDDING_DIM), jnp.float32),
        pltpu.VMEM((_BLK * EMBEDDING_DIM,), jnp.float32),
        pltpu.VMEM((_BLK * EMBEDDING_DIM,), jnp.float32),
        pltpu.SemaphoreType.DMA,
        pltpu.SemaphoreType.DMA,
        pltpu.SemaphoreType.DMA,
    ],
    compiler_params=pltpu.CompilerParams(
        use_tc_tiling_on_sc=False, needs_layout_passes=False),
)
def _gather_kernel(ids_hbm, table_hbm, out_hbm, idx_all, idb0, idb1,
                   gb0, gb1, tb0, tb1, sem_g, sem_s0, sem_s1):
    wid = lax.axis_index("s") * 2 + lax.axis_index("c")
    base = wid * _PER_W
    pltpu.sync_copy(ids_hbm.at[pl.ds(base, _PER_W)], idx_all)

    iv50 = lax.iota(jnp.int32, 16) * HIST
    iv128 = lax.iota(jnp.int32, 16) * _BLK
    idbufs = (idb0, idb1)
    gbufs = (gb0, gb1)
    tbufs = (tb0, tb1)
    sems = (sem_s0, sem_s1)

    def stage(t, pb):
        q = t % 4
        h = t // 4
        off = q * (_BLK * HIST) + h
        for j0 in range(8):
            v = plsc.load_gather(idx_all, [iv50 + (off + j0 * 16 * HIST)])
            idbufs[pb][pl.ds(j0 * 16, 16)] = v

    def g_start(pb):
        pltpu.make_async_copy(
            table_hbm.at[idbufs[pb]], gbufs[pb], sem_g).start()

    def g_wait():
        pltpu.make_async_copy(
            table_hbm.at[idbufs[0]], gbufs[0], sem_g).wait()

    def transpose(pb):
        g = gbufs[pb]
        tb = tbufs[pb]

        @plsc.parallel_loop(0, _BLK, unroll=8)
        def _(j):
            v0 = g[j, pl.ds(0, 16)]
            v1 = g[j, pl.ds(16, 16)]
            plsc.store_scatter(tb, [iv128 + j], v0)
            plsc.store_scatter(tb, [iv128 + (j + 16 * _BLK)], v1)

    def s_start(t, pb):
        q = t % 4
        h = t // 4
        b1 = wid * 4 + q
        for d1 in range(4):
            pltpu.make_async_copy(
                tbufs[pb].at[pl.ds(d1 * 8 * _BLK, 8 * _BLK)],
                out_hbm.at[h, d1, b1], sems[pb]).start()

    def s_wait(pb):
        for _ in range(4):
            pltpu.make_async_copy(
                tbufs[pb].at[pl.ds(0, 8 * _BLK)],
                out_hbm.at[0, 0, 0], sems[pb]).wait()

    stage(0, 0)
    g_start(0)
    stage(1, 1)
    g_start(1)
    g_wait()
    transpose(0)
    s_start(0, 0)
    stage(2, 0)
    g_start(0)
    g_wait()
    transpose(1)
    s_start(1, 1)

    def pair(gi, carry):
        t = 2 + 2 * gi
        stage(t + 1, 1)
        g_start(1)
        g_wait()
        s_wait(0)
        transpose(0)
        s_start(t, 0)
        stage(t + 2, 0)
        g_start(0)
        g_wait()
        s_wait(1)
        transpose(1)
        s_start(t + 1, 1)
        return carry

    lax.fori_loop(0, (_NBLK - 4) // 2, pair, 0)

    stage(_NBLK - 1, 1)
    g_start(1)
    g_wait()
    s_wait(0)
    transpose(0)
    s_start(_NBLK - 2, 0)
    g_wait()
    s_wait(1)
    transpose(1)
    s_start(_NBLK - 1, 1)
    s_wait(0)
    s_wait(1)


def kernel(token_ids, weights):
    flat_ids = token_ids.reshape(_TOTAL)
    out = _gather_kernel(flat_ids, weights)
    out = out.reshape(HIST, 4, BATCH // _BLK, 8, _BLK)
    return out.transpose(2, 4, 0, 1, 3).reshape(BATCH, HIST, EMBEDDING_DIM)
